# baseline (speedup 1.0000x reference)
import functools

import jax
import jax.numpy as jnp
from jax.experimental import pallas as pl

_T = 4096


def _vq_body(z_ref, cb_ref, cbt_ref, zq_ref, idx_ref):
    zb = z_ref[0]
    cb = cb_ref[...]
    s = jax.lax.dot_general(
        cb, zb, (((1,), (0,)), ((), ())),
        preferred_element_type=jnp.float32,
    )
    cn = jnp.sum(cb * cb, axis=1, keepdims=True)
    dist = cn - 2.0 * s
    idx = jnp.argmin(dist, axis=0)
    idx_ref[0, 0] = idx
    onehot = (jax.lax.broadcasted_iota(jnp.int32, dist.shape, 0)
              == idx[None, :]).astype(jnp.float32)
    zq_ref[0] = jax.lax.dot_general(
        cbt_ref[...], onehot, (((1,), (0,)), ((), ())),
        preferred_element_type=jnp.float32,
    )


@functools.partial(jax.jit, static_argnames=())
def kernel(z_e, codebook):
    B, D, H, W = z_e.shape
    K = codebook.shape[0]
    HW = H * W
    z3 = z_e.astype(jnp.bfloat16).reshape(B, D, HW)
    cbt = codebook.T

    zq3, idx3 = pl.pallas_call(
        _vq_body,
        grid=(B,),
        in_specs=[
            pl.BlockSpec((1, D, _T), lambda b: (b, 0, 0)),
            pl.BlockSpec((K, D), lambda b: (0, 0)),
            pl.BlockSpec((D, K), lambda b: (0, 0)),
        ],
        out_specs=[
            pl.BlockSpec((1, D, _T), lambda b: (b, 0, 0)),
            pl.BlockSpec((1, 1, _T), lambda b: (b, 0, 0)),
        ],
        out_shape=[
            jax.ShapeDtypeStruct((B, D, HW), jnp.float32),
            jax.ShapeDtypeStruct((B, 1, HW), jnp.int32),
        ],
    )(z3, codebook, cbt)

    return zq3.reshape(B, D, H, W), idx3.reshape(B, H, W)


# speedup vs baseline: 1.0704x; 1.0704x over previous
import functools

import jax
import jax.numpy as jnp
from jax.experimental import pallas as pl

_T = 4096


def _vq_body(z_ref, cb_ref, cbt_ref, zq_ref, idx_ref):
    zb = z_ref[0]
    cb = cb_ref[...]
    s = jax.lax.dot_general(
        cb, zb, (((1,), (0,)), ((), ())),
        preferred_element_type=jnp.float32,
    )
    cn = jnp.sum(cb * cb, axis=1, keepdims=True)
    dist = cn - 2.0 * s
    idx = jnp.argmin(dist, axis=0)
    idx_ref[0, 0] = idx
    onehot = (jax.lax.broadcasted_iota(jnp.int32, dist.shape, 0)
              == idx[None, :]).astype(jnp.float32)
    zq_ref[0] = jax.lax.dot_general(
        cbt_ref[...], onehot, (((1,), (0,)), ((), ())),
        preferred_element_type=jnp.float32,
    )


@functools.partial(jax.jit, static_argnames=())
def kernel(z_e, codebook):
    B, D, H, W = z_e.shape
    K = codebook.shape[0]
    HW = H * W
    z3 = z_e.reshape(B, D, HW)
    cbt = codebook.T

    zq3, idx3 = pl.pallas_call(
        _vq_body,
        grid=(B,),
        in_specs=[
            pl.BlockSpec((1, D, _T), lambda b: (b, 0, 0)),
            pl.BlockSpec((K, D), lambda b: (0, 0)),
            pl.BlockSpec((D, K), lambda b: (0, 0)),
        ],
        out_specs=[
            pl.BlockSpec((1, D, _T), lambda b: (b, 0, 0)),
            pl.BlockSpec((1, 1, _T), lambda b: (b, 0, 0)),
        ],
        out_shape=[
            jax.ShapeDtypeStruct((B, D, HW), jnp.float32),
            jax.ShapeDtypeStruct((B, 1, HW), jnp.int32),
        ],
    )(z3, codebook, cbt)

    return zq3.reshape(B, D, H, W), idx3.reshape(B, H, W)


# speedup vs baseline: 1.0722x; 1.0017x over previous
import functools

import jax
import jax.numpy as jnp
from jax.experimental import pallas as pl
from jax.experimental.pallas import tpu as pltpu

_T = 4096


def _vq_body(z_ref, cb_ref, zq_ref, idx_ref):
    zb = z_ref[0]
    cb = cb_ref[...]
    s = jax.lax.dot_general(
        cb, zb, (((1,), (0,)), ((), ())),
        preferred_element_type=jnp.float32,
    )
    cn = jnp.sum(cb * cb, axis=1, keepdims=True)
    dist = cn - 2.0 * s
    idx = jnp.argmin(dist, axis=0)
    idx_ref[0, 0] = idx

    onehot = (jax.lax.broadcasted_iota(jnp.int32, dist.shape, 0)
              == idx[None, :]).astype(jnp.float32)
    zq_ref[0] = jax.lax.dot_general(
        cb, onehot, (((0,), (0,)), ((), ())),
        preferred_element_type=jnp.float32,
    )


@functools.partial(jax.jit, static_argnames=())
def kernel(z_e, codebook):
    B, D, H, W = z_e.shape
    K = codebook.shape[0]
    HW = H * W
    z3 = z_e.reshape(B, D, HW)

    zq3, idx3 = pl.pallas_call(
        _vq_body,
        grid=(B,),
        in_specs=[
            pl.BlockSpec((1, D, _T), lambda b: (b, 0, 0)),
            pl.BlockSpec((K, D), lambda b: (0, 0)),
        ],
        out_specs=[
            pl.BlockSpec((1, D, _T), lambda b: (b, 0, 0)),
            pl.BlockSpec((1, 1, _T), lambda b: (b, 0, 0)),
        ],
        out_shape=[
            jax.ShapeDtypeStruct((B, D, HW), jnp.float32),
            jax.ShapeDtypeStruct((B, 1, HW), jnp.int32),
        ],
    )(z3, codebook)

    return zq3.reshape(B, D, H, W), idx3.reshape(B, H, W)


# speedup vs baseline: 1.1955x; 1.1150x over previous
import functools

import jax
import jax.numpy as jnp
from jax.experimental import pallas as pl
from jax.experimental.pallas import tpu as pltpu

_T = 4096


def _vq_body(z_ref, cb_ref, zq_ref, idx_ref):
    zb = z_ref[0]
    cb = cb_ref[...]
    cn = jnp.sum(cb * cb, axis=1, keepdims=True) * 0.5
    cn_hi = cn.astype(jnp.bfloat16).astype(jnp.float32)
    r1 = cn - cn_hi
    cn_mid = r1.astype(jnp.bfloat16).astype(jnp.float32)
    cn_lo = r1 - cn_mid
    cba = jnp.concatenate([cb, cn_hi, cn_mid, cn_lo], axis=1)
    zba = jnp.concatenate(
        [zb, jnp.full((3, zb.shape[1]), -1.0, zb.dtype)], axis=0)
    s = jax.lax.dot_general(
        cba, zba, (((1,), (0,)), ((), ())),
        preferred_element_type=jnp.float32,
    )
    idx = jnp.argmax(s, axis=0)
    idx_ref[0, 0] = idx

    onehot = (jax.lax.broadcasted_iota(jnp.int32, s.shape, 0)
              == idx[None, :]).astype(jnp.float32)
    zq_ref[0] = jax.lax.dot_general(
        cb, onehot, (((0,), (0,)), ((), ())),
        preferred_element_type=jnp.float32,
    )


@functools.partial(jax.jit, static_argnames=())
def kernel(z_e, codebook):
    B, D, H, W = z_e.shape
    K = codebook.shape[0]
    HW = H * W
    z3 = z_e.reshape(B, D, HW)

    zq3, idx3 = pl.pallas_call(
        _vq_body,
        grid=(B,),
        in_specs=[
            pl.BlockSpec((1, D, _T), lambda b: (b, 0, 0)),
            pl.BlockSpec((K, D), lambda b: (0, 0)),
        ],
        out_specs=[
            pl.BlockSpec((1, D, _T), lambda b: (b, 0, 0)),
            pl.BlockSpec((1, 1, _T), lambda b: (b, 0, 0)),
        ],
        out_shape=[
            jax.ShapeDtypeStruct((B, D, HW), jnp.float32),
            jax.ShapeDtypeStruct((B, 1, HW), jnp.int32),
        ],
    )(z3, codebook)

    return zq3.reshape(B, D, H, W), idx3.reshape(B, H, W)


# speedup vs baseline: 1.2257x; 1.0252x over previous
import functools

import jax
import jax.numpy as jnp
from jax.experimental import pallas as pl

_T = 4096
_BB = 2


def _vq_body(z_ref, cb_ref, zq_ref, idx_ref):
    cb = cb_ref[...]
    cn = jnp.sum(cb * cb, axis=1, keepdims=True) * 0.5
    cn_hi = cn.astype(jnp.bfloat16).astype(jnp.float32)
    r1 = cn - cn_hi
    cn_mid = r1.astype(jnp.bfloat16).astype(jnp.float32)
    cn_lo = r1 - cn_mid
    cba = jnp.concatenate([cb, cn_hi, cn_mid, cn_lo], axis=1)
    for b2 in range(_BB):
        zb = z_ref[b2]
        zba = jnp.concatenate(
            [zb, jnp.full((3, zb.shape[1]), -1.0, zb.dtype)], axis=0)
        s = jax.lax.dot_general(
            cba, zba, (((1,), (0,)), ((), ())),
            preferred_element_type=jnp.float32,
        )
        idx = jnp.argmax(s, axis=0)
        idx_ref[b2, 0] = idx
        onehot = (jax.lax.broadcasted_iota(jnp.int32, s.shape, 0)
                  == idx[None, :]).astype(jnp.float32)
        zq_ref[b2] = jax.lax.dot_general(
            cb, onehot, (((0,), (0,)), ((), ())),
            preferred_element_type=jnp.float32,
        )


@functools.partial(jax.jit, static_argnames=())
def kernel(z_e, codebook):
    B, D, H, W = z_e.shape
    K = codebook.shape[0]
    HW = H * W
    z3 = z_e.reshape(B, D, HW)

    zq3, idx3 = pl.pallas_call(
        _vq_body,
        grid=(B // _BB,),
        in_specs=[
            pl.BlockSpec((_BB, D, _T), lambda b: (b, 0, 0)),
            pl.BlockSpec((K, D), lambda b: (0, 0)),
        ],
        out_specs=[
            pl.BlockSpec((_BB, D, _T), lambda b: (b, 0, 0)),
            pl.BlockSpec((_BB, 1, _T), lambda b: (b, 0, 0)),
        ],
        out_shape=[
            jax.ShapeDtypeStruct((B, D, HW), jnp.float32),
            jax.ShapeDtypeStruct((B, 1, HW), jnp.int32),
        ],
    )(z3, codebook)

    return zq3.reshape(B, D, H, W), idx3.reshape(B, H, W)
